# Initial kernel scaffold; baseline (speedup 1.0000x reference)
import jax
import jax.numpy as jnp
from jax.experimental import pallas as pl


def kernel(positions, pe):
    raise NotImplementedError("write your pallas kernel here")


# speedup vs baseline: 4.7480x; 4.7480x over previous
import functools

import jax
import jax.numpy as jnp
from jax import lax
from jax.experimental import pallas as pl
from jax.experimental.pallas import tpu as pltpu
from jax.experimental.pallas import tpu_sc as plsc


@functools.lru_cache(maxsize=None)
def _make_gather(B, V, D):
    info = plsc.get_sparse_core_info()
    NC, NS = info.num_cores, info.num_subcores
    NW = NC * NS

    G = 128
    C = 512
    K = C // G

    assert B % (NW * C) == 0, (B, NW, C)
    b_per_w = B // NW
    n_chunks = b_per_w // C

    mesh = plsc.VectorSubcoreMesh(core_axis_name="c", subcore_axis_name="s")

    @functools.partial(
        pl.kernel,
        mesh=mesh,
        compiler_params=pltpu.CompilerParams(use_tc_tiling_on_sc=False),
        out_type=jax.ShapeDtypeStruct((B, D), jnp.float32),
        scratch_types=[
            pltpu.VMEM((K, G), jnp.int32),
            pltpu.VMEM((C, D), jnp.float32),
            pltpu.SemaphoreType.DMA,
        ],
    )
    def gather_kernel(table_hbm, idx_hbm, out_hbm, idx_v, rows_v, sem):
        wid = lax.axis_index("s") * NC + lax.axis_index("c")
        row0 = wid * (b_per_w // G)

        def body(c, carry):
            off = wid * b_per_w + c * C
            pltpu.sync_copy(idx_hbm.at[pl.ds(row0 + c * K, K)], idx_v)
            copies = [
                pltpu.async_copy(
                    table_hbm.at[idx_v.at[j]],
                    rows_v.at[pl.ds(j * G, G)],
                    sem,
                )
                for j in range(K)
            ]
            for cp in copies:
                cp.wait()
            pltpu.sync_copy(rows_v, out_hbm.at[pl.ds(off, C)])
            return carry

        lax.fori_loop(0, n_chunks, body, 0)

    return gather_kernel


def kernel(positions, pe):
    B = positions.shape[0] * positions.shape[1]
    D = pe.shape[1]
    idx = positions.reshape(B // 128, 128).astype(jnp.int32)
    pe = pe.astype(jnp.float32)
    out = _make_gather(B, pe.shape[0], D)(pe, idx)
    return out.reshape(*positions.shape, D)


# speedup vs baseline: 5.1050x; 1.0752x over previous
import functools

import jax
import jax.numpy as jnp
from jax import lax
from jax.experimental import pallas as pl
from jax.experimental.pallas import tpu as pltpu
from jax.experimental.pallas import tpu_sc as plsc


@functools.lru_cache(maxsize=None)
def _make_gather(B, V, D):
    info = plsc.get_sparse_core_info()
    NC, NS = info.num_cores, info.num_subcores
    NW = NC * NS

    G = 128
    C = 512
    K = C // G

    assert B % (NW * C) == 0, (B, NW, C)
    b_per_w = B // NW
    n_chunks = b_per_w // C
    assert n_chunks % 2 == 0 and n_chunks >= 4

    mesh = plsc.VectorSubcoreMesh(core_axis_name="c", subcore_axis_name="s")

    @functools.partial(
        pl.kernel,
        mesh=mesh,
        compiler_params=pltpu.CompilerParams(use_tc_tiling_on_sc=False),
        out_type=jax.ShapeDtypeStruct((B, D), jnp.float32),
        scratch_types=[
            pltpu.VMEM((2, K, G), jnp.int32),
            pltpu.VMEM((2, C, D), jnp.float32),
            pltpu.SemaphoreType.DMA((2,)),
            pltpu.SemaphoreType.DMA,
            pltpu.SemaphoreType.DMA((2,)),
        ],
    )
    def gather_kernel(table_hbm, idx_hbm, out_hbm, idx_v, rows_v, sem_i, sem_g, sem_o):
        wid = lax.axis_index("s") * NC + lax.axis_index("c")
        row0 = wid * (b_per_w // G)
        base = wid * b_per_w

        def start_idx(c, b):
            c = lax.min(c, n_chunks - 1) if not isinstance(c, int) else min(c, n_chunks - 1)
            return pltpu.async_copy(
                idx_hbm.at[pl.ds(row0 + c * K, K)], idx_v.at[b], sem_i.at[b]
            )

        def wait_idx(b):
            pltpu.make_async_copy(
                idx_hbm.at[pl.ds(row0, K)], idx_v.at[b], sem_i.at[b]
            ).wait()

        def run_gathers(b):
            copies = [
                pltpu.async_copy(
                    table_hbm.at[idx_v.at[b].at[j]],
                    rows_v.at[b].at[pl.ds(j * G, G)],
                    sem_g,
                )
                for j in range(K)
            ]
            for cp in copies:
                cp.wait()

        def start_store(c, b):
            return pltpu.async_copy(
                rows_v.at[b], out_hbm.at[pl.ds(base + c * C, C)], sem_o.at[b]
            )

        def wait_store(b):
            pltpu.make_async_copy(
                rows_v.at[b], out_hbm.at[pl.ds(base, C)], sem_o.at[b]
            ).wait()

        start_idx(0, 0)
        start_idx(1, 1)
        for b in (0, 1):
            wait_idx(b)
            run_gathers(b)
            start_store(b, b)
            start_idx(2 + b, b)

        def body(t, carry):
            for b in (0, 1):
                c = 2 * t + b
                wait_idx(b)
                wait_store(b)
                run_gathers(b)
                start_store(c, b)
                start_idx(c + 2, b)
            return carry

        lax.fori_loop(1, n_chunks // 2, body, 0)

        for b in (0, 1):
            wait_store(b)
            wait_idx(b)

    return gather_kernel


def kernel(positions, pe):
    B = positions.shape[0] * positions.shape[1]
    D = pe.shape[1]
    idx = positions.reshape(B // 128, 128).astype(jnp.int32)
    pe = pe.astype(jnp.float32)
    out = _make_gather(B, pe.shape[0], D)(pe, idx)
    return out.reshape(*positions.shape, D)
